# baseline (speedup 1.0000x reference)
import functools

import jax
import jax.numpy as jnp
from jax import lax
from jax.experimental import pallas as pl
from jax.experimental.pallas import tpu as pltpu
from jax.experimental.pallas import tpu_sc as plsc

MPE_ = 128
DIM_ = 64
B_ = 16
L_ = 200
LPAD_ = 224
NW_ = 32
ROWS_PER_W_ = 256 // NW_
TROW_ = 256 * DIM_


def _build_body(x_hbm, y_hbm, bb_hbm, tab_hbm, ax_hbm, ay_hbm,
                xv, yv, bbv, axb, ayb, tb, sem_t):
    cid = lax.axis_index("c")
    sid = lax.axis_index("s")
    w = sid * 2 + cid

    pltpu.sync_copy(y_hbm, yv)
    pltpu.sync_copy(x_hbm.at[pl.ds(w * (ROWS_PER_W_ * DIM_), ROWS_PER_W_ * DIM_)], xv)

    @pl.when(w < B_)
    def _():
        for c in range(4):
            pltpu.sync_copy(bb_hbm.at[c * B_ + w], bbv.at[c])
        for k in range(LPAD_ // 16):
            sl = pl.ds(k * 16, 16)
            axb[sl] = (bbv[0, sl] * float(MPE_) + bbv[2, sl] * float(MPE_)) * 0.5
            ayb[sl] = (bbv[1, sl] * float(MPE_) + bbv[3, sl] * float(MPE_)) * 0.5
        pltpu.sync_copy(axb, ax_hbm.at[w])
        pltpu.sync_copy(ayb, ay_hbm.at[w])

    for r in range(ROWS_PER_W_):
        slot = r % 2
        if r >= 2:
            pltpu.make_async_copy(tab_hbm.at[pl.ds(0, 256)],
                                  tb.at[pl.ds(slot * 256, 256)],
                                  sem_t).wait()
        xr = [xv[pl.ds(r * DIM_ + c * 16, 16)] for c in range(4)]

        def iyb(iy, carry, slot=slot, xr=xr):
            row = slot * 256 + iy
            for c in range(4):
                tb[row, pl.ds(c * 16, 16)] = (
                    yv[pl.ds(iy * DIM_ + c * 16, 16)] + xr[c])
            return carry

        lax.fori_loop(0, 256, iyb, 0)
        pltpu.async_copy(
            tb.at[pl.ds(slot * 256, 256)],
            tab_hbm.at[pl.ds((w * ROWS_PER_W_ + r) * 256, 256)], sem_t)

    for s in range(2):
        pltpu.make_async_copy(tab_hbm.at[pl.ds(0, 256)],
                              tb.at[pl.ds(s * 256, 256)], sem_t).wait()


NUNIT_ = L_ // 2
GSZ_ = 4
GROUPS_ = NUNIT_ // GSZ_
GROWS_ = GSZ_ * L_
ISTR_ = 208
NIDX_ = NUNIT_ * ISTR_


def _sc_body(tab_hbm, ax_hbm, ay_hbm, out_hbm, axv, ayv, idxv, rows,
             sem_g, sem_o):
    cid = lax.axis_index("c")
    sid = lax.axis_index("s")
    b = sid
    i0 = cid * NUNIT_

    pltpu.sync_copy(ax_hbm.at[b], axv)
    pltpu.sync_copy(ay_hbm.at[b], ayv)

    def unit(u, carry):
        i = i0 + u
        axi = axv[pl.ds(i, 16)][0]
        ayi = ayv[pl.ds(i, 16)][0]
        for k in range(13):
            aj = axv[pl.ds(k * 16, 16)]
            dx = axi - aj + float(MPE_)
            ix = jnp.clip(dx, 0.0, float(2 * MPE_ - 1)).astype(jnp.int32)
            bj = ayv[pl.ds(k * 16, 16)]
            dy = ayi - bj + float(MPE_)
            iy = jnp.clip(dy, 0.0, float(2 * MPE_ - 1)).astype(jnp.int32)
            idxv[pl.ds(u * ISTR_ + k * 16, 16)] = ix * 256 + iy
        return carry

    lax.fori_loop(0, NUNIT_, unit, 0)

    wbase = (b * L_ + i0) * L_

    def group(g, carry):
        off = (g % 2) * GROWS_

        @pl.when(g >= 2)
        def _():
            pltpu.make_async_copy(
                out_hbm.at[pl.ds(0, GROWS_)], rows.at[pl.ds(off, GROWS_)],
                sem_o).wait()

        cps = []
        for s in range(GSZ_):
            jb = (g * GSZ_ + s) * ISTR_
            dst = off + s * L_
            cps.append(pltpu.async_copy(
                tab_hbm.at[idxv.at[pl.ds(jb, 104)]],
                rows.at[pl.ds(dst, 104)], sem_g))
            cps.append(pltpu.async_copy(
                tab_hbm.at[idxv.at[pl.ds(jb + 104, 96)]],
                rows.at[pl.ds(dst + 104, 96)], sem_g))
        for cp in cps:
            cp.wait()
        pltpu.async_copy(rows.at[pl.ds(off, GROWS_)],
                         out_hbm.at[pl.ds(wbase + g * GROWS_, GROWS_)], sem_o)
        return carry

    lax.fori_loop(0, GROUPS_, group, 0)
    for _ in range(2):
        pltpu.make_async_copy(out_hbm.at[pl.ds(0, GROWS_)],
                              rows.at[pl.ds(0, GROWS_)], sem_o).wait()


FI_ = 8


def _fmt_body(x_ref, o_ref):
    x = x_ref[...]
    for ii in range(FI_):
        blk = x[ii * 100:(ii + 1) * 100, :]
        te = jnp.transpose(blk[:, :DIM_])
        to = jnp.transpose(blk[:, DIM_:])
        z = jnp.stack([te, to], axis=-1).reshape(DIM_, L_)
        o_ref[0, ii] = z


_fmt_call = pl.pallas_call(
    _fmt_body,
    grid=(B_, L_ // FI_),
    in_specs=[pl.BlockSpec((FI_ * 100, 128),
                           lambda b, i: (b * (L_ // FI_) + i, 0))],
    out_specs=pl.BlockSpec((1, FI_, DIM_, L_), lambda b, i: (b, i, 0, 0)),
    out_shape=jax.ShapeDtypeStruct((B_, L_, DIM_, L_), jnp.float32),
)


@functools.lru_cache(maxsize=1)
def _get_calls():
    mesh = plsc.VectorSubcoreMesh(core_axis_name="c", subcore_axis_name="s")
    params = pltpu.CompilerParams(use_tc_tiling_on_sc=False)
    build = functools.partial(
        pl.kernel,
        mesh=mesh,
        compiler_params=params,
        out_type=(
            jax.ShapeDtypeStruct((256 * 256, DIM_), jnp.float32),
            jax.ShapeDtypeStruct((B_, LPAD_), jnp.float32),
            jax.ShapeDtypeStruct((B_, LPAD_), jnp.float32),
        ),
        scratch_types=[
            pltpu.VMEM((ROWS_PER_W_ * DIM_,), jnp.float32),
            pltpu.VMEM((256 * DIM_,), jnp.float32),
            pltpu.VMEM((4, LPAD_), jnp.float32),
            pltpu.VMEM((LPAD_,), jnp.float32),
            pltpu.VMEM((LPAD_,), jnp.float32),
            pltpu.VMEM((2 * 256, DIM_), jnp.float32),
            pltpu.SemaphoreType.DMA,
        ],
    )(_build_body)
    gather = functools.partial(
        pl.kernel,
        mesh=mesh,
        compiler_params=params,
        out_type=jax.ShapeDtypeStruct((B_ * L_ * L_, DIM_), jnp.float32),
        scratch_types=[
            pltpu.VMEM((LPAD_,), jnp.float32),
            pltpu.VMEM((LPAD_,), jnp.float32),
            pltpu.VMEM((NIDX_,), jnp.int32),
            pltpu.VMEM((2 * GROWS_, DIM_), jnp.float32),
            pltpu.SemaphoreType.DMA,
            pltpu.SemaphoreType.DMA,
        ],
    )(_sc_body)
    return build, gather


def kernel(gt_bboxes, x_emb, y_emb):
    bbT = jnp.transpose(
        jnp.pad(gt_bboxes, ((0, 0), (0, LPAD_ - L_), (0, 0))), (2, 0, 1)
    ).reshape(4 * B_, LPAD_)
    build, gather = _get_calls()
    tab, ax, ay = build(x_emb.reshape(-1), y_emb.reshape(-1), bbT)
    out = gather(tab, ax, ay)
    out_t = _fmt_call(out.reshape(B_ * L_ * 100, 128))
    return jnp.swapaxes(out_t, 2, 3)


# speedup vs baseline: 14.3434x; 14.3434x over previous
import functools

import jax
import jax.numpy as jnp
from jax import lax
from jax.experimental import pallas as pl
from jax.experimental.pallas import tpu as pltpu
from jax.experimental.pallas import tpu_sc as plsc

MPE_ = 128
DIM_ = 64
B_ = 16
L_ = 200
LPAD_ = 224
NW_ = 32
ROWS_PER_W_ = 256 // NW_
TROW_ = 256 * DIM_


def _build_body(x_hbm, y_hbm, bb_hbm, tab_hbm, ax_hbm, ay_hbm,
                xv, yv, bbv, axb, ayb, tb, sem_t):
    cid = lax.axis_index("c")
    sid = lax.axis_index("s")
    w = sid * 2 + cid

    pltpu.sync_copy(y_hbm, yv)
    pltpu.sync_copy(x_hbm.at[pl.ds(w * (ROWS_PER_W_ * DIM_), ROWS_PER_W_ * DIM_)], xv)

    @pl.when(w < B_)
    def _():
        for c in range(4):
            pltpu.sync_copy(bb_hbm.at[c * B_ + w], bbv.at[c])
        for k in range(LPAD_ // 16):
            sl = pl.ds(k * 16, 16)
            axb[sl] = (bbv[0, sl] * float(MPE_) + bbv[2, sl] * float(MPE_)) * 0.5
            ayb[sl] = (bbv[1, sl] * float(MPE_) + bbv[3, sl] * float(MPE_)) * 0.5
        pltpu.sync_copy(axb, ax_hbm.at[w])
        pltpu.sync_copy(ayb, ay_hbm.at[w])

    for r in range(ROWS_PER_W_):
        slot = r % 2
        if r >= 2:
            pltpu.make_async_copy(tab_hbm.at[pl.ds(0, 256)],
                                  tb.at[pl.ds(slot * 256, 256)],
                                  sem_t).wait()
        xr = [xv[pl.ds(r * DIM_ + c * 16, 16)] for c in range(4)]

        def iyb(iy, carry, slot=slot, xr=xr):
            row = slot * 256 + iy
            for c in range(4):
                tb[row, pl.ds(c * 16, 16)] = (
                    yv[pl.ds(iy * DIM_ + c * 16, 16)] + xr[c])
            return carry

        lax.fori_loop(0, 256, iyb, 0)
        pltpu.async_copy(
            tb.at[pl.ds(slot * 256, 256)],
            tab_hbm.at[pl.ds((w * ROWS_PER_W_ + r) * 256, 256)], sem_t)

    for s in range(2):
        pltpu.make_async_copy(tab_hbm.at[pl.ds(0, 256)],
                              tb.at[pl.ds(s * 256, 256)], sem_t).wait()


NUNIT_ = L_ // 2
ISTR_ = 208
NIDX_ = NUNIT_ * ISTR_


def _sc_body(tab_hbm, ax_hbm, ay_hbm, out_hbm, axv, ayv, idxv, stg, obuf,
             sem_g, sem_o):
    cid = lax.axis_index("c")
    sid = lax.axis_index("s")
    b = sid
    i0 = cid * NUNIT_

    pltpu.sync_copy(ax_hbm.at[b], axv)
    pltpu.sync_copy(ay_hbm.at[b], ayv)

    def unit(u, carry):
        i = i0 + u
        axi = axv[pl.ds(i, 16)][0]
        ayi = ayv[pl.ds(i, 16)][0]
        for k in range(13):
            aj = axv[pl.ds(k * 16, 16)]
            dx = axi - aj + float(MPE_)
            ix = jnp.clip(dx, 0.0, float(2 * MPE_ - 1)).astype(jnp.int32)
            bj = ayv[pl.ds(k * 16, 16)]
            dy = ayi - bj + float(MPE_)
            iy = jnp.clip(dy, 0.0, float(2 * MPE_ - 1)).astype(jnp.int32)
            idxv[pl.ds(u * ISTR_ + k * 16, 16)] = ix * 256 + iy
        return carry

    lax.fori_loop(0, NUNIT_, unit, 0)

    iota16 = lax.iota(jnp.int32, 16)
    basecc = [(cc * 16 + iota16) * L_ for cc in range(4)]

    def _issue(u, s):
        jb = u * ISTR_
        so = s * ISTR_
        pltpu.async_copy(tab_hbm.at[idxv.at[pl.ds(jb, 104)]],
                         stg.at[pl.ds(so, 104)], sem_g)
        pltpu.async_copy(tab_hbm.at[idxv.at[pl.ds(jb + 104, 96)]],
                         stg.at[pl.ds(so + 104, 96)], sem_g)

    _issue(0, 0)
    _issue(1, 1)

    OSZ = DIM_ * L_

    def pair(up, carry):
        for s in (0, 1):
            u = 2 * up + s
            so = s * ISTR_
            oo = s * OSZ

            @pl.when(u >= 2)
            def _():
                pltpu.make_async_copy(out_hbm.at[0].at[0],
                                      obuf.at[pl.ds(oo, OSZ)], sem_o).wait()

            pltpu.make_async_copy(tab_hbm.at[pl.ds(0, 104)],
                                  stg.at[pl.ds(so, 104)], sem_g).wait()
            pltpu.make_async_copy(tab_hbm.at[pl.ds(0, 96)],
                                  stg.at[pl.ds(so + 104, 96)], sem_g).wait()

            def jloop(jq, c2, so=so, oo=oo):
                for q in range(4):
                    j = jq * 4 + q
                    row = so + j
                    od = oo + j
                    for cc in range(4):
                        vals = stg[row, pl.ds(cc * 16, 16)]
                        plsc.store_scatter(obuf, [basecc[cc] + od], vals)
                return c2

            lax.fori_loop(0, L_ // 4, jloop, 0)

            @pl.when(u < NUNIT_ - 2)
            def _(u=u, s=s):
                _issue(u + 2, s)

            pltpu.async_copy(obuf.at[pl.ds(oo, OSZ)],
                             out_hbm.at[b].at[i0 + u], sem_o)
        return carry

    lax.fori_loop(0, NUNIT_ // 2, pair, 0)
    for s in (0, 1):
        pltpu.make_async_copy(out_hbm.at[0].at[0],
                              obuf.at[pl.ds(s * OSZ, OSZ)], sem_o).wait()


@functools.lru_cache(maxsize=1)
def _get_calls():
    mesh = plsc.VectorSubcoreMesh(core_axis_name="c", subcore_axis_name="s")
    params = pltpu.CompilerParams(use_tc_tiling_on_sc=False)
    build = functools.partial(
        pl.kernel,
        mesh=mesh,
        compiler_params=params,
        out_type=(
            jax.ShapeDtypeStruct((256 * 256, DIM_), jnp.float32),
            jax.ShapeDtypeStruct((B_, LPAD_), jnp.float32),
            jax.ShapeDtypeStruct((B_, LPAD_), jnp.float32),
        ),
        scratch_types=[
            pltpu.VMEM((ROWS_PER_W_ * DIM_,), jnp.float32),
            pltpu.VMEM((256 * DIM_,), jnp.float32),
            pltpu.VMEM((4, LPAD_), jnp.float32),
            pltpu.VMEM((LPAD_,), jnp.float32),
            pltpu.VMEM((LPAD_,), jnp.float32),
            pltpu.VMEM((2 * 256, DIM_), jnp.float32),
            pltpu.SemaphoreType.DMA,
        ],
    )(_build_body)
    gparams = pltpu.CompilerParams(use_tc_tiling_on_sc=False,
                                   needs_layout_passes=False)
    gather = functools.partial(
        pl.kernel,
        mesh=mesh,
        compiler_params=gparams,
        out_type=jax.ShapeDtypeStruct((B_, L_, DIM_ * L_), jnp.float32),
        scratch_types=[
            pltpu.VMEM((LPAD_,), jnp.float32),
            pltpu.VMEM((LPAD_,), jnp.float32),
            pltpu.VMEM((NIDX_,), jnp.int32),
            pltpu.VMEM((2 * ISTR_, DIM_), jnp.float32),
            pltpu.VMEM((2 * DIM_ * L_,), jnp.float32),
            pltpu.SemaphoreType.DMA,
            pltpu.SemaphoreType.DMA,
        ],
    )(_sc_body)
    return build, gather


def kernel(gt_bboxes, x_emb, y_emb):
    bbT = jnp.transpose(
        jnp.pad(gt_bboxes, ((0, 0), (0, LPAD_ - L_), (0, 0))), (2, 0, 1)
    ).reshape(4 * B_, LPAD_)
    build, gather = _get_calls()
    tab, ax, ay = build(x_emb.reshape(-1), y_emb.reshape(-1), bbT)
    out = gather(tab, ax, ay)
    return jnp.swapaxes(out.reshape(B_, L_, DIM_, L_), 2, 3)
